# baseline (speedup 1.0000x reference)
import functools

import numpy as np
import jax
import jax.numpy as jnp
from jax import lax
from jax.experimental import pallas as pl
from jax.experimental.pallas import tpu as pltpu
from jax.experimental.pallas import tpu_sc as plsc


def _pe_table(seq_len, dm):
    pos = np.arange(seq_len, dtype=np.float32)[:, None]
    col = np.arange(dm)[None, :]
    expn = ((col // 2).astype(np.float32) * (2.0 / dm)).astype(np.float32)
    base = np.power(pos / 10000.0, expn, dtype=np.float32)
    pe = np.where(col % 2 == 0, np.sin(base), np.cos(base)).astype(np.float32)
    return jnp.asarray(pe.reshape(-1))


def _sc_embed(text, table, pe):
    nb, seq_len = text.shape
    dm = table.shape[1]
    n_tok = nb * seq_len

    mesh = plsc.VectorSubcoreMesh(core_axis_name="c", subcore_axis_name="s")
    info = plsc.get_sparse_core_info()
    ncores = info.num_cores
    nw = info.num_cores * info.num_subcores
    ch = seq_len // nw

    sub = 2
    rows = ch // sub
    n_st = nb * sub

    @functools.partial(
        pl.kernel,
        mesh=mesh,
        out_type=jax.ShapeDtypeStruct((n_tok, dm), jnp.float32),
        scratch_types=[
            pltpu.VMEM((nb, ch), jnp.int32),
            pltpu.VMEM((ch * dm,), jnp.float32),
            pltpu.VMEM((2, rows, dm), jnp.float32),
            pltpu.SemaphoreType.DMA,
            pltpu.SemaphoreType.DMA,
            pltpu.SemaphoreType.DMA,
            pltpu.SemaphoreType.DMA,
        ],
    )
    def body(idx_hbm, table_hbm, pe_hbm, out_hbm, idx_v, pe_v, bufs, g0, g1, o0, o1):
        wid = lax.axis_index("s") * ncores + lax.axis_index("c")
        gsem = (g0, g1)
        osem = (o0, o1)

        def idx_ref(st):
            return idx_v.at[st // sub, pl.ds((st % sub) * rows, rows)]

        pltpu.sync_copy(idx_hbm.at[0, pl.ds(wid * ch, ch)], idx_v.at[0])
        gathers = {}
        outs = {}
        gathers[0] = pltpu.async_copy(table_hbm.at[idx_ref(0)], bufs.at[0], g0)
        for b in range(1, nb):
            pltpu.sync_copy(idx_hbm.at[b, pl.ds(wid * ch, ch)], idx_v.at[b])
        pltpu.sync_copy(pe_hbm.at[pl.ds(wid * ch * dm, ch * dm)], pe_v)

        def add_pe(buf, pe_base):
            @plsc.parallel_loop(0, rows, 1, unroll=4)
            def _row(r):
                for g in range(dm // 16):
                    plsc.addupdate(
                        buf.at[r, pl.ds(g * 16, 16)],
                        pe_v[pl.ds(pe_base + r * dm + g * 16, 16)],
                    )

        for st in range(n_st):
            s = st % 2
            if st + 1 < n_st:
                if st >= 1:
                    outs[st - 1].wait()
                gathers[st + 1] = pltpu.async_copy(
                    table_hbm.at[idx_ref(st + 1)], bufs.at[(st + 1) % 2], gsem[(st + 1) % 2]
                )
            gathers[st].wait()
            add_pe(bufs.at[s], (st % sub) * rows * dm)
            outs[st] = pltpu.async_copy(
                bufs.at[s],
                out_hbm.at[pl.ds((st // sub) * seq_len + wid * ch + (st % sub) * rows, rows)],
                osem[s],
            )
        outs[n_st - 2].wait()
        outs[n_st - 1].wait()

    return body(text, table, pe)


def kernel(text, embed_table):
    b, seq_len = text.shape
    dm = embed_table.shape[1]
    pe = _pe_table(seq_len, dm)
    out = _sc_embed(text, embed_table, pe)
    return out.reshape(b, seq_len, dm)


# speedup vs baseline: 1.1276x; 1.1276x over previous
import functools

import numpy as np
import jax
import jax.numpy as jnp
from jax import lax
from jax.experimental import pallas as pl
from jax.experimental.pallas import tpu as pltpu
from jax.experimental.pallas import tpu_sc as plsc


def _pe_table(seq_len, dm):
    pos = np.arange(seq_len, dtype=np.float32)[:, None]
    col = np.arange(dm)[None, :]
    expn = ((col // 2).astype(np.float32) * (2.0 / dm)).astype(np.float32)
    base = np.power(pos / 10000.0, expn, dtype=np.float32)
    pe = np.where(col % 2 == 0, np.sin(base), np.cos(base)).astype(np.float32)
    return jnp.asarray(pe.reshape(-1))


def _sc_embed(text, table, pe):
    nb, seq_len = text.shape
    dm = table.shape[1]
    n_tok = nb * seq_len

    mesh = plsc.VectorSubcoreMesh(core_axis_name="c", subcore_axis_name="s")
    info = plsc.get_sparse_core_info()
    ncores = info.num_cores
    nw = info.num_cores * info.num_subcores
    ch = seq_len // nw

    sub = 1
    rows = ch // sub
    n_st = nb * sub

    @functools.partial(
        pl.kernel,
        mesh=mesh,
        out_type=jax.ShapeDtypeStruct((n_tok, dm), jnp.float32),
        scratch_types=[
            pltpu.VMEM((nb, ch), jnp.int32),
            pltpu.VMEM((ch * dm,), jnp.float32),
            pltpu.VMEM((2, rows, dm), jnp.float32),
            pltpu.SemaphoreType.DMA,
            pltpu.SemaphoreType.DMA,
            pltpu.SemaphoreType.DMA,
            pltpu.SemaphoreType.DMA,
        ],
    )
    def body(idx_hbm, table_hbm, pe_hbm, out_hbm, idx_v, pe_v, bufs, g0, g1, o0, o1):
        wid = lax.axis_index("s") * ncores + lax.axis_index("c")
        gsem = (g0, g1)
        osem = (o0, o1)

        def idx_ref(st):
            return idx_v.at[st // sub, pl.ds((st % sub) * rows, rows)]

        pltpu.sync_copy(idx_hbm.at[0, pl.ds(wid * ch, ch)], idx_v.at[0])
        gathers = {}
        outs = {}
        gathers[0] = pltpu.async_copy(table_hbm.at[idx_ref(0)], bufs.at[0], g0)
        for b in range(1, nb):
            pltpu.sync_copy(idx_hbm.at[b, pl.ds(wid * ch, ch)], idx_v.at[b])
        pltpu.sync_copy(pe_hbm.at[pl.ds(wid * ch * dm, ch * dm)], pe_v)

        def add_pe(buf, pe_base):
            @plsc.parallel_loop(0, rows, 1, unroll=4)
            def _row(r):
                for g in range(dm // 16):
                    plsc.addupdate(
                        buf.at[r, pl.ds(g * 16, 16)],
                        pe_v[pl.ds(pe_base + r * dm + g * 16, 16)],
                    )

        for st in range(n_st):
            s = st % 2
            if st + 1 < n_st:
                if st >= 1:
                    outs[st - 1].wait()
                gathers[st + 1] = pltpu.async_copy(
                    table_hbm.at[idx_ref(st + 1)], bufs.at[(st + 1) % 2], gsem[(st + 1) % 2]
                )
            gathers[st].wait()
            add_pe(bufs.at[s], (st % sub) * rows * dm)
            outs[st] = pltpu.async_copy(
                bufs.at[s],
                out_hbm.at[pl.ds((st // sub) * seq_len + wid * ch + (st % sub) * rows, rows)],
                osem[s],
            )
        outs[n_st - 2].wait()
        outs[n_st - 1].wait()

    return body(text, table, pe)


def kernel(text, embed_table):
    b, seq_len = text.shape
    dm = embed_table.shape[1]
    pe = _pe_table(seq_len, dm)
    out = _sc_embed(text, embed_table, pe)
    return out.reshape(b, seq_len, dm)


# speedup vs baseline: 1.2325x; 1.0930x over previous
import functools

import numpy as np
import jax
import jax.numpy as jnp
from jax import lax
from jax.experimental import pallas as pl
from jax.experimental.pallas import tpu as pltpu
from jax.experimental.pallas import tpu_sc as plsc


def _pe_table(seq_len, dm):
    pos = np.arange(seq_len, dtype=np.float32)[:, None]
    col = np.arange(dm)[None, :]
    expn = ((col // 2).astype(np.float32) * (2.0 / dm)).astype(np.float32)
    base = np.power(pos / 10000.0, expn, dtype=np.float32)
    pe = np.where(col % 2 == 0, np.sin(base), np.cos(base)).astype(np.float32)
    v = pe.reshape(-1).view(np.uint32)
    bf = ((v + 0x7FFF + ((v >> 16) & 1)) >> 16).astype(np.uint32)
    blk = bf.reshape(-1, 2, 16)
    packed = blk[:, 0, :] | (blk[:, 1, :] << 16)
    return jnp.asarray(packed.reshape(-1))


def _sc_embed(text, table, pe):
    nb, seq_len = text.shape
    dm = table.shape[1]
    n_tok = nb * seq_len

    mesh = plsc.VectorSubcoreMesh(core_axis_name="c", subcore_axis_name="s")
    info = plsc.get_sparse_core_info()
    ncores = info.num_cores
    nw = info.num_cores * info.num_subcores
    ch = seq_len // nw

    sub = 1
    rows = ch // sub
    n_st = nb * sub

    @functools.partial(
        pl.kernel,
        mesh=mesh,
        out_type=jax.ShapeDtypeStruct((n_tok, dm), jnp.float32),
        scratch_types=[
            pltpu.VMEM((nb, ch), jnp.int32),
            pltpu.VMEM((ch * dm // 2,), jnp.uint32),
            pltpu.VMEM((2, rows, dm), jnp.float32),
            pltpu.SemaphoreType.DMA,
            pltpu.SemaphoreType.DMA,
            pltpu.SemaphoreType.DMA,
            pltpu.SemaphoreType.DMA,
        ],
    )
    def body(idx_hbm, table_hbm, pe_hbm, out_hbm, idx_v, pe_v, bufs, g0, g1, o0, o1):
        wid = lax.axis_index("s") * ncores + lax.axis_index("c")
        gsem = (g0, g1)
        osem = (o0, o1)

        def idx_ref(st):
            return idx_v.at[st // sub, pl.ds((st % sub) * rows, rows)]

        pltpu.sync_copy(idx_hbm.at[0, pl.ds(wid * ch, ch)], idx_v.at[0])
        gathers = {}
        outs = {}
        gathers[0] = pltpu.async_copy(table_hbm.at[idx_ref(0)], bufs.at[0], g0)
        for b in range(1, nb):
            pltpu.sync_copy(idx_hbm.at[b, pl.ds(wid * ch, ch)], idx_v.at[b])
        pe_off = pl.multiple_of(wid * (ch * dm // 2), 8)
        pltpu.sync_copy(pe_hbm.at[pl.ds(pe_off, ch * dm // 2)], pe_v)

        def add_pe(buf, pe_base):
            hmask = jnp.uint32(0xFFFF0000)

            @plsc.parallel_loop(0, rows, 1, unroll=4)
            def _row(r):
                for g in range(dm // 32):
                    w = pe_v[pl.ds(pe_base + (r * dm + g * 32) // 2, 16)]
                    lo = lax.bitcast_convert_type(w << 16, jnp.float32)
                    hi = lax.bitcast_convert_type(w & hmask, jnp.float32)
                    plsc.addupdate(buf.at[r, pl.ds(g * 32, 16)], lo)
                    plsc.addupdate(buf.at[r, pl.ds(g * 32 + 16, 16)], hi)

        for st in range(n_st):
            s = st % 2
            if st + 1 < n_st:
                if st >= 1:
                    outs[st - 1].wait()
                gathers[st + 1] = pltpu.async_copy(
                    table_hbm.at[idx_ref(st + 1)], bufs.at[(st + 1) % 2], gsem[(st + 1) % 2]
                )
            gathers[st].wait()
            add_pe(bufs.at[s], (st % sub) * rows * dm // 2)
            outs[st] = pltpu.async_copy(
                bufs.at[s],
                out_hbm.at[pl.ds((st // sub) * seq_len + wid * ch + (st % sub) * rows, rows)],
                osem[s],
            )
        outs[n_st - 2].wait()
        outs[n_st - 1].wait()

    return body(text, table, pe)


def kernel(text, embed_table):
    b, seq_len = text.shape
    dm = embed_table.shape[1]
    pe = _pe_table(seq_len, dm)
    out = _sc_embed(text, embed_table, pe)
    return out.reshape(b, seq_len, dm)
